# Initial kernel scaffold; baseline (speedup 1.0000x reference)
import functools

import jax
import jax.numpy as jnp
from jax import lax
from jax.experimental import pallas as pl
from jax.experimental.pallas import tpu as pltpu
from jax.experimental.pallas import tpu_sc as plsc

_H = 128
_NC = 2
_NS = 16
_NW = _NC * _NS
_K = 2
_C = _K * 128
_NBUF = 2


def _lookup_body(ids_hbm, table_hbm, out_hbm, idx_v, rows_v,
                 sem_idx, sem_g, sem_o0, sem_o1):
    rows_total = ids_hbm.shape[0]
    rows_per_w = rows_total // _NW
    nchunks = rows_per_w // _K
    wid = lax.axis_index("s") * _NC + lax.axis_index("c")
    row_base = wid * rows_per_w
    out_base = row_base * 128

    sem_o = (sem_o0, sem_o1)

    def idx_copy(g, b):
        return pltpu.make_async_copy(
            ids_hbm.at[pl.ds(row_base + g * _K, _K)],
            idx_v.at[pl.ds(b * _K, _K)], sem_idx)

    def gather_copy(g, b, j):
        return pltpu.make_async_copy(
            table_hbm.at[idx_v.at[b * _K + j]],
            rows_v.at[b, pl.ds(j * 128, 128)], sem_g)

    def out_copy(g, b):
        return pltpu.make_async_copy(
            rows_v.at[b],
            out_hbm.at[pl.ds(out_base + g * _C, _C)], sem_o[b])

    idx_copy(0, 0).start()

    @pl.loop(0, nchunks, step=_NBUF)
    def _(gg):
        for b in range(_NBUF):
            g = gg + b

            @pl.when(g + 1 < nchunks)
            def _():
                idx_copy(g + 1, 1 - b).start()

            idx_copy(g, b).wait()

            @pl.when(g >= _NBUF)
            def _():
                out_copy(g - _NBUF, b).wait()

            for j in range(_K):
                gather_copy(g, b, j).start()
            for j in range(_K):
                gather_copy(g, b, j).wait()
            out_copy(g, b).start()

    out_copy(nchunks - 2, 0).wait()
    out_copy(nchunks - 1, 1).wait()


def kernel(input_ids, W_orig, W_special, W_prompt_prefix):
    batch, seq = input_ids.shape
    n = batch * seq
    table = jnp.concatenate([W_orig, W_special, W_prompt_prefix], axis=0)
    ids2 = input_ids.reshape(n // 128, 128)

    mesh = plsc.VectorSubcoreMesh(core_axis_name="c", subcore_axis_name="s",
                                  num_cores=_NC, num_subcores=_NS)
    run = pl.kernel(
        _lookup_body,
        out_type=jax.ShapeDtypeStruct((n, _H), jnp.float32),
        mesh=mesh,
        scratch_types=[
            pltpu.VMEM((_NBUF * _K, 128), jnp.int32),
            pltpu.VMEM((_NBUF, _C, _H), jnp.float32),
            pltpu.SemaphoreType.DMA,
            pltpu.SemaphoreType.DMA,
            pltpu.SemaphoreType.DMA,
            pltpu.SemaphoreType.DMA,
        ],
    )
    out = run(ids2, table)
    return out.reshape(batch, seq, _H)


# speedup vs baseline: 25.2532x; 25.2532x over previous
import functools

import jax
import jax.numpy as jnp
from jax import lax
from jax.experimental import pallas as pl
from jax.experimental.pallas import tpu as pltpu
from jax.experimental.pallas import tpu_sc as plsc

_H = 128
_NC = 2
_NS = 16
_NW = _NC * _NS
_K = 2
_C = _K * 128
_NBUF = 2


def _lookup_body(ids_hbm, table_hbm, out_hbm, idx_v, rows_v,
                 sem_i0, sem_i1, sem_g, sem_o0, sem_o1):
    rows_total = ids_hbm.shape[0]
    rows_per_w = rows_total // _NW
    nchunks = rows_per_w // _K
    wid = lax.axis_index("s") * _NC + lax.axis_index("c")
    row_base = wid * rows_per_w
    out_base = row_base * 128

    sem_i = (sem_i0, sem_i1)
    sem_o = (sem_o0, sem_o1)

    def idx_copy(g, b):
        return pltpu.make_async_copy(
            ids_hbm.at[pl.ds(row_base + g * _K, _K)],
            idx_v.at[pl.ds(b * _K, _K)], sem_i[b])

    def gather_copy(g, b, j):
        return pltpu.make_async_copy(
            table_hbm.at[idx_v.at[b * _K + j]],
            rows_v.at[b, pl.ds(j * 128, 128)], sem_g)

    def out_copy(g, b):
        return pltpu.make_async_copy(
            rows_v.at[b],
            out_hbm.at[pl.ds(out_base + g * _C, _C)], sem_o[b])

    idx_copy(0, 0).start()

    @pl.loop(0, nchunks, step=_NBUF)
    def _(gg):
        for b in range(_NBUF):
            g = gg + b

            @pl.when(g + 1 < nchunks)
            def _():
                idx_copy(g + 1, 1 - b).start()

            idx_copy(g, b).wait()

            @pl.when(g >= _NBUF)
            def _():
                out_copy(g - _NBUF, b).wait()

            for j in range(_K):
                gather_copy(g, b, j).start()
            for j in range(_K):
                gather_copy(g, b, j).wait()
            out_copy(g, b).start()

    out_copy(nchunks - 2, 0).wait()
    out_copy(nchunks - 1, 1).wait()


def kernel(input_ids, W_orig, W_special, W_prompt_prefix):
    batch, seq = input_ids.shape
    n = batch * seq
    table = jnp.concatenate([W_orig, W_special, W_prompt_prefix], axis=0)
    ids2 = input_ids.reshape(n // 128, 128)

    mesh = plsc.VectorSubcoreMesh(core_axis_name="c", subcore_axis_name="s",
                                  num_cores=_NC, num_subcores=_NS)
    run = pl.kernel(
        _lookup_body,
        out_type=jax.ShapeDtypeStruct((n, _H), jnp.float32),
        mesh=mesh,
        scratch_types=[
            pltpu.VMEM((_NBUF * _K, 128), jnp.int32),
            pltpu.VMEM((_NBUF, _C, _H), jnp.float32),
            pltpu.SemaphoreType.DMA,
            pltpu.SemaphoreType.DMA,
            pltpu.SemaphoreType.DMA,
            pltpu.SemaphoreType.DMA,
            pltpu.SemaphoreType.DMA,
        ],
    )
    out = run(ids2, table)
    return out.reshape(batch, seq, _H)


# speedup vs baseline: 25.3156x; 1.0025x over previous
import jax
import jax.numpy as jnp
from jax import lax
from jax.experimental import pallas as pl
from jax.experimental.pallas import tpu as pltpu
from jax.experimental.pallas import tpu_sc as plsc

_H = 128
_NC = 2
_NS = 16
_NW = _NC * _NS
_CH = 128
_NBUF = 4


def _lookup_body(ids_hbm, table_hbm, out_hbm, idx_v, rows_v,
                 sem_g0, sem_g1, sem_g2, sem_g3,
                 sem_o0, sem_o1, sem_o2, sem_o3):
    rows_total = ids_hbm.shape[0]
    rows_per_w = rows_total // _NW
    nchunks = rows_per_w
    wid = lax.axis_index("s") * _NC + lax.axis_index("c")
    row_base = wid * rows_per_w
    out_base = row_base * _CH

    sem_g = (sem_g0, sem_g1, sem_g2, sem_g3)
    sem_o = (sem_o0, sem_o1, sem_o2, sem_o3)

    def gather_copy(g, b):
        return pltpu.make_async_copy(
            table_hbm.at[idx_v.at[g]], rows_v.at[b], sem_g[b])

    def out_copy(g, b):
        return pltpu.make_async_copy(
            rows_v.at[b],
            out_hbm.at[pl.ds(out_base + g * _CH, _CH)], sem_o[b])

    pltpu.sync_copy(ids_hbm.at[pl.ds(row_base, rows_per_w)], idx_v)
    gather_copy(0, 0).start()

    @pl.loop(0, nchunks, step=_NBUF)
    def _(gg):
        for b in range(_NBUF):
            g = gg + b
            nb = (b + 1) % _NBUF

            @pl.when(g >= _NBUF - 1)
            def _():
                out_copy(g - (_NBUF - 1), nb).wait()

            @pl.when(g + 1 < nchunks)
            def _():
                gather_copy(g + 1, nb).start()

            gather_copy(g, b).wait()
            out_copy(g, b).start()

    for t in range(_NBUF - 1, 0, -1):
        out_copy(nchunks - t, (nchunks - t) % _NBUF).wait()


def kernel(input_ids, W_orig, W_special, W_prompt_prefix):
    batch, seq = input_ids.shape
    n = batch * seq
    table = jnp.concatenate([W_orig, W_special, W_prompt_prefix], axis=0)
    ids2 = input_ids.reshape(n // 128, 128)
    rows_per_w = (n // 128) // _NW

    mesh = plsc.VectorSubcoreMesh(core_axis_name="c", subcore_axis_name="s",
                                  num_cores=_NC, num_subcores=_NS)
    run = pl.kernel(
        _lookup_body,
        out_type=jax.ShapeDtypeStruct((n, _H), jnp.float32),
        mesh=mesh,
        scratch_types=[
            pltpu.VMEM((rows_per_w, 128), jnp.int32),
            pltpu.VMEM((_NBUF, _CH, _H), jnp.float32),
            pltpu.SemaphoreType.DMA,
            pltpu.SemaphoreType.DMA,
            pltpu.SemaphoreType.DMA,
            pltpu.SemaphoreType.DMA,
            pltpu.SemaphoreType.DMA,
            pltpu.SemaphoreType.DMA,
            pltpu.SemaphoreType.DMA,
            pltpu.SemaphoreType.DMA,
        ],
    )
    out = run(ids2, table)
    return out.reshape(batch, seq, _H)


# speedup vs baseline: 25.3347x; 1.0008x over previous
import jax
import jax.numpy as jnp
from jax import lax
from jax.experimental import pallas as pl
from jax.experimental.pallas import tpu as pltpu
from jax.experimental.pallas import tpu_sc as plsc

_H = 128
_NC = 2
_NS = 16
_NW = _NC * _NS
_CH = 128
_NBUF = 5
_GD = 3


def _lookup_body(ids_hbm, table_hbm, out_hbm, idx_v, rows_v,
                 sem_g0, sem_g1, sem_g2, sem_g3, sem_g4,
                 sem_o0, sem_o1, sem_o2, sem_o3, sem_o4):
    rows_total = ids_hbm.shape[0]
    rows_per_w = rows_total // _NW
    nchunks = rows_per_w
    wid = lax.axis_index("s") * _NC + lax.axis_index("c")
    row_base = wid * rows_per_w
    out_base = row_base * _CH

    sem_g = (sem_g0, sem_g1, sem_g2, sem_g3, sem_g4)
    sem_o = (sem_o0, sem_o1, sem_o2, sem_o3, sem_o4)

    def gather_copy(g, b):
        return pltpu.make_async_copy(
            table_hbm.at[idx_v.at[g]], rows_v.at[b], sem_g[b])

    def out_copy(g, b):
        return pltpu.make_async_copy(
            rows_v.at[b],
            out_hbm.at[pl.ds(out_base + g * _CH, _CH)], sem_o[b])

    pltpu.sync_copy(ids_hbm.at[pl.ds(row_base, rows_per_w)], idx_v)
    for p in range(_GD - 1):
        gather_copy(p, p).start()

    ahead = _GD - 1
    @pl.loop(0, nchunks, step=_NBUF)
    def _(gg):
        for b in range(_NBUF):
            g = gg + b
            nb = (b + ahead) % _NBUF

            @pl.when(g >= _NBUF - ahead)
            def _():
                out_copy(g - (_NBUF - ahead), nb).wait()

            @pl.when(g + ahead < nchunks)
            def _():
                gather_copy(g + ahead, nb).start()

            gather_copy(g, b).wait()
            out_copy(g, b).start()

    for t in range(_NBUF - ahead, 0, -1):
        out_copy(nchunks - t, (nchunks - t) % _NBUF).wait()


def kernel(input_ids, W_orig, W_special, W_prompt_prefix):
    batch, seq = input_ids.shape
    n = batch * seq
    table = jnp.concatenate([W_orig, W_special, W_prompt_prefix], axis=0)
    ids2 = input_ids.reshape(n // 128, 128)
    rows_per_w = (n // 128) // _NW

    mesh = plsc.VectorSubcoreMesh(core_axis_name="c", subcore_axis_name="s",
                                  num_cores=_NC, num_subcores=_NS)
    run = pl.kernel(
        _lookup_body,
        out_type=jax.ShapeDtypeStruct((n, _H), jnp.float32),
        mesh=mesh,
        scratch_types=[
            pltpu.VMEM((rows_per_w, 128), jnp.int32),
            pltpu.VMEM((_NBUF, _CH, _H), jnp.float32),
            pltpu.SemaphoreType.DMA,
            pltpu.SemaphoreType.DMA,
            pltpu.SemaphoreType.DMA,
            pltpu.SemaphoreType.DMA,
            pltpu.SemaphoreType.DMA,
            pltpu.SemaphoreType.DMA,
            pltpu.SemaphoreType.DMA,
            pltpu.SemaphoreType.DMA,
            pltpu.SemaphoreType.DMA,
            pltpu.SemaphoreType.DMA,
        ],
    )
    out = run(ids2, table)
    return out.reshape(batch, seq, _H)
